# Initial kernel scaffold; baseline (speedup 1.0000x reference)
import jax
import jax.numpy as jnp
from jax.experimental import pallas as pl


def kernel(x, pe):
    raise NotImplementedError("write your pallas kernel here")


# speedup vs baseline: 2.0818x; 2.0818x over previous
import jax
import jax.numpy as jnp
from jax.experimental import pallas as pl

_D_X = 1024
_DIM = 128
_S_BLK = 512


def _concat_body(x_ref, pe_ref, o_ref):
    o_ref[:, :, :_D_X] = x_ref[...]
    o_ref[:, :, _D_X:] = pe_ref[...][None, :, :]


def kernel(x, pe):
    b, s, d_x = x.shape
    dim = pe.shape[1]
    grid = (b, s // _S_BLK)
    return pl.pallas_call(
        _concat_body,
        grid=grid,
        in_specs=[
            pl.BlockSpec((1, _S_BLK, d_x), lambda i, j: (i, j, 0)),
            pl.BlockSpec((_S_BLK, dim), lambda i, j: (j, 0)),
        ],
        out_specs=pl.BlockSpec((1, _S_BLK, d_x + dim), lambda i, j: (i, j, 0)),
        out_shape=jax.ShapeDtypeStruct((b, s, d_x + dim), x.dtype),
    )(x, pe)


# speedup vs baseline: 2.2310x; 1.0717x over previous
import jax
import jax.numpy as jnp
from jax.experimental import pallas as pl

_D_X = 1024
_DIM = 128
_S_BLK = 1024


def _concat_body(x_ref, pe_ref, o_ref):
    o_ref[:, :, :_D_X] = x_ref[...]
    o_ref[:, :, _D_X:] = pe_ref[...][None, :, :]


def kernel(x, pe):
    b, s, d_x = x.shape
    dim = pe.shape[1]
    grid = (b, s // _S_BLK)
    return pl.pallas_call(
        _concat_body,
        grid=grid,
        in_specs=[
            pl.BlockSpec((1, _S_BLK, d_x), lambda i, j: (i, j, 0)),
            pl.BlockSpec((_S_BLK, dim), lambda i, j: (j, 0)),
        ],
        out_specs=pl.BlockSpec((1, _S_BLK, d_x + dim), lambda i, j: (i, j, 0)),
        out_shape=jax.ShapeDtypeStruct((b, s, d_x + dim), x.dtype),
    )(x, pe)


# speedup vs baseline: 2.2806x; 1.0222x over previous
import jax
import jax.numpy as jnp
from jax.experimental import pallas as pl

_D_X = 1024
_DIM = 128
_S_BLK = 2048


def _concat_body(x_ref, pe_ref, o_ref):
    o_ref[:, :, :_D_X] = x_ref[...]
    o_ref[:, :, _D_X:] = pe_ref[...][None, :, :]


def kernel(x, pe):
    b, s, d_x = x.shape
    dim = pe.shape[1]
    grid = (b, s // _S_BLK)
    return pl.pallas_call(
        _concat_body,
        grid=grid,
        in_specs=[
            pl.BlockSpec((1, _S_BLK, d_x), lambda i, j: (i, j, 0)),
            pl.BlockSpec((_S_BLK, dim), lambda i, j: (j, 0)),
        ],
        out_specs=pl.BlockSpec((1, _S_BLK, d_x + dim), lambda i, j: (i, j, 0)),
        out_shape=jax.ShapeDtypeStruct((b, s, d_x + dim), x.dtype),
    )(x, pe)


# speedup vs baseline: 2.4000x; 1.0523x over previous
import jax
import jax.numpy as jnp
from jax.experimental import pallas as pl

_D_X = 1024
_S_BLK = 2048


def _concat_body(x_ref, pe_ref, o_ref):
    j = pl.program_id(1)
    o_ref[:, :, :_D_X] = x_ref[...]
    o_ref[:, :, _D_X:] = pe_ref[pl.ds(j * _S_BLK, _S_BLK), :][None, :, :]


def kernel(x, pe):
    b, s, d_x = x.shape
    size, dim = pe.shape
    grid = (b, s // _S_BLK)
    return pl.pallas_call(
        _concat_body,
        grid=grid,
        in_specs=[
            pl.BlockSpec((1, _S_BLK, d_x), lambda i, j: (i, j, 0)),
            pl.BlockSpec((size, dim), lambda i, j: (0, 0)),
        ],
        out_specs=pl.BlockSpec((1, _S_BLK, d_x + dim), lambda i, j: (i, j, 0)),
        out_shape=jax.ShapeDtypeStruct((b, s, d_x + dim), x.dtype),
    )(x, pe)
